# Initial kernel scaffold; baseline (speedup 1.0000x reference)
import jax
import jax.numpy as jnp
from jax.experimental import pallas as pl


def kernel(x_A, x_B, edge_index_ab, edge_feat_ab, edge_index_ba, edge_feat_ba, ab_msg_W1, ab_msg_b1, ab_msg_W2, ab_msg_b2, ab_udt_W1, ab_udt_b1, ab_udt_W2, ab_udt_b2, ba_msg_W1, ba_msg_b1, ba_msg_W2, ba_msg_b2, ba_udt_W1, ba_udt_b1, ba_udt_W2, ba_udt_b2):
    raise NotImplementedError("write your pallas kernel here")


# speedup vs baseline: 1.6063x; 1.6063x over previous
import functools

import jax
import jax.numpy as jnp
from jax import lax
from jax.experimental import pallas as pl
from jax.experimental.pallas import tpu as pltpu
from jax.experimental.pallas import tpu_sc as plsc

N = 10000
E = 320000
D_IN = 128
D_EDGE = 16
MSG = 32
HID = 16

NW = 32
LANES = 16
GCHUNK = 2000
SCHUNK = 8000
NEG = float("-inf")


def _g_body(xa_ref, xb_ref, wa_ref, ba_ref, wb_ref, bb_ref, ga_ref, gb_ref):
    ga_ref[...] = lax.dot_general(
        xa_ref[...], wa_ref[...], (((1,), (1,)), ((), ())),
        preferred_element_type=jnp.float32) + ba_ref[...]
    gb_ref[...] = lax.dot_general(
        xb_ref[...], wb_ref[...], (((1,), (1,)), ((), ())),
        preferred_element_type=jnp.float32) + bb_ref[...]


def _node_proj(x_A, x_B, W1x_ab, b1_ab, W1x_ba, b1_ba):
    return pl.pallas_call(
        _g_body,
        out_shape=[jax.ShapeDtypeStruct((N, HID), jnp.float32)] * 2,
    )(x_A, x_B, W1x_ab, b1_ab.reshape(1, HID), W1x_ba, b1_ba.reshape(1, HID))


def _msg_body(ef_ref, gat_ref, we_ref, w2_ref, b2_ref, mt_ref):
    h = lax.dot_general(ef_ref[...], we_ref[...], (((1,), (1,)), ((), ())),
                        preferred_element_type=jnp.float32)
    h = jnp.maximum(h + gat_ref[...], 0.0)
    mt_ref[...] = lax.dot_general(
        w2_ref[...], h, (((1,), (1,)), ((), ())),
        preferred_element_type=jnp.float32) + b2_ref[...]


def _msg_mlp(efeat, gat, W1e, W2, b2):
    be = 16000
    grid = E // be
    return pl.pallas_call(
        _msg_body,
        grid=(grid,),
        in_specs=[
            pl.BlockSpec((be, D_EDGE), lambda j: (j, 0)),
            pl.BlockSpec((be, HID), lambda j: (j, 0)),
            pl.BlockSpec((HID, D_EDGE), lambda j: (0, 0)),
            pl.BlockSpec((MSG, HID), lambda j: (0, 0)),
            pl.BlockSpec((MSG, 1), lambda j: (0, 0)),
        ],
        out_specs=pl.BlockSpec((MSG, be), lambda j: (0, j)),
        out_shape=jax.ShapeDtypeStruct((MSG, E), jnp.float32),
    )(efeat, gat, W1e, W2, b2.reshape(MSG, 1))


def _udt_body(x_ref, rt_ref, w1x_ref, w1r_ref, b1_ref, w2_ref, b2_ref, o_ref):
    rt = rt_ref[...]
    rt = jnp.where(jnp.isneginf(rt), 0.0, rt)
    h = lax.dot_general(x_ref[...], w1x_ref[...], (((1,), (1,)), ((), ())),
                        preferred_element_type=jnp.float32)
    h = h + lax.dot_general(rt, w1r_ref[...], (((0,), (1,)), ((), ())),
                            preferred_element_type=jnp.float32)
    h = jnp.maximum(h + b1_ref[...], 0.0)
    o_ref[...] = lax.dot_general(
        h, w2_ref[...], (((1,), (1,)), ((), ())),
        preferred_element_type=jnp.float32) + b2_ref[...]


def _udt_mlp(x_dst, r_t, uW1x, uW1r, ub1, uW2, ub2):
    return pl.pallas_call(
        _udt_body,
        out_shape=jax.ShapeDtypeStruct((N, D_IN), jnp.float32),
    )(x_dst, r_t, uW1x, uW1r, ub1.reshape(1, HID), uW2, ub2.reshape(1, D_IN))


_MESH = None


def _mesh():
    global _MESH
    if _MESH is None:
        _MESH = plsc.VectorSubcoreMesh(core_axis_name="c", subcore_axis_name="s")
    return _MESH


def _gather_kernel(ga_hbm, sa_hbm, gb_hbm, sb_hbm, oa_hbm, ob_hbm,
                   idx_v, rows_v, sem):
    wid = lax.axis_index("s") * 2 + lax.axis_index("c")
    per_w = E // NW
    nchunk = per_w // GCHUNK

    for g_hbm, s_hbm, o_hbm in ((ga_hbm, sa_hbm, oa_hbm),
                                (gb_hbm, sb_hbm, ob_hbm)):
        def body(k, _, g_hbm=g_hbm, s_hbm=s_hbm, o_hbm=o_hbm):
            base = wid * per_w + k * GCHUNK
            pltpu.sync_copy(s_hbm.at[pl.ds(base, GCHUNK)], idx_v)
            pltpu.async_copy(g_hbm.at[idx_v], rows_v, sem).wait()
            pltpu.sync_copy(rows_v, o_hbm.at[pl.ds(base, GCHUNK)])
            return _
        lax.fori_loop(0, nchunk, body, None)


def _sc_gather(g_ab, src_ab, g_ba, src_ba):
    k = pl.kernel(
        _gather_kernel,
        out_type=[jax.ShapeDtypeStruct((E, HID), jnp.float32)] * 2,
        mesh=_mesh(),
        compiler_params=pltpu.CompilerParams(use_tc_tiling_on_sc=False),
        scratch_types=[
            pltpu.VMEM((GCHUNK,), jnp.int32),
            pltpu.VMEM((GCHUNK, HID), jnp.float32),
            pltpu.SemaphoreType.DMA,
        ],
    )
    return k(g_ab, src_ab, g_ba, src_ba)


def _scatter_kernel(ma_hbm, da_hbm, mb_hbm, db_hbm, ra_hbm, rb_hbm,
                    dst_v, val_v, acc_v, bm_ref, sem):
    wid = lax.axis_index("s") * 2 + lax.axis_index("c")
    nchunk = E // SCHUNK
    nstep = SCHUNK // LANES
    zero = jnp.zeros((LANES,), jnp.int32)
    iota = lax.iota(jnp.int32, LANES)
    pw2 = jnp.left_shift(jnp.ones((LANES,), jnp.int32), iota)
    zvec = jnp.zeros((LANES,), jnp.int32)
    i0 = jnp.asarray(0, jnp.int32)

    def update(i, mask):
        idx = dst_v[pl.ds(i * LANES, LANES)]
        val = val_v[0, pl.ds(i * LANES, LANES)]
        old = plsc.load_gather(acc_v, [zero, idx], mask=mask)
        new = jnp.maximum(old, val)
        plsc.store_scatter(acc_v, [zero, idx], new, mask=mask)
        chk = plsc.load_gather(acc_v, [zero, idx], mask=mask)
        lost = mask & (chk < new)
        bm = jnp.sum(jnp.where(lost, pw2, zvec))
        bm_ref[i] = bm
        return bm

    for m_hbm, d_hbm, r_hbm in ((ma_hbm, da_hbm, ra_hbm),
                                (mb_hbm, db_hbm, rb_hbm)):
        def init(i, _):
            acc_v[0, pl.ds(i * LANES, LANES)] = jnp.full((LANES,), NEG,
                                                         jnp.float32)
            return _
        lax.fori_loop(0, N // LANES, init, None)

        def chunk(k, _, m_hbm=m_hbm, d_hbm=d_hbm):
            base = k * SCHUNK
            pltpu.sync_copy(d_hbm.at[pl.ds(base, SCHUNK)], dst_v)
            pltpu.sync_copy(m_hbm.at[pl.ds(wid, 1), pl.ds(base, SCHUNK)],
                            val_v)

            def pass1(i, tot):
                return tot + update(i, jnp.ones((LANES,), jnp.bool_))
            tot = lax.fori_loop(0, nstep, pass1, i0)

            def retry(p, tot):
                def live():
                    def step2(i, t2):
                        bm = bm_ref[i]

                        def redo():
                            mask = jnp.bitwise_and(jnp.right_shift(
                                jnp.full((LANES,), bm, jnp.int32), iota),
                                1) != 0
                            return update(i, mask)
                        return t2 + lax.cond(bm > 0, redo, lambda: i0)
                    return lax.fori_loop(0, nstep, step2, i0)
                return lax.cond(tot > 0, live, lambda: i0)
            lax.fori_loop(0, LANES - 1, retry, tot)
            return _
        lax.fori_loop(0, nchunk, chunk, None)
        pltpu.sync_copy(acc_v, r_hbm.at[pl.ds(wid, 1)])


def _sc_segment_max(m_t_ab, dst_ab, m_t_ba, dst_ba):
    k = pl.kernel(
        _scatter_kernel,
        out_type=[jax.ShapeDtypeStruct((MSG, N), jnp.float32)] * 2,
        mesh=_mesh(),
        compiler_params=pltpu.CompilerParams(use_tc_tiling_on_sc=False,
                                             needs_layout_passes=False),
        scratch_types=[
            pltpu.VMEM((SCHUNK,), jnp.int32),
            pltpu.VMEM((1, SCHUNK), jnp.float32),
            pltpu.VMEM((1, N), jnp.float32),
            pltpu.SMEM((SCHUNK // LANES,), jnp.int32),
            pltpu.SemaphoreType.DMA,
        ],
    )
    return k(m_t_ab, dst_ab, m_t_ba, dst_ba)


def kernel(x_A, x_B, edge_index_ab, edge_feat_ab, edge_index_ba, edge_feat_ba,
           ab_msg_W1, ab_msg_b1, ab_msg_W2, ab_msg_b2,
           ab_udt_W1, ab_udt_b1, ab_udt_W2, ab_udt_b2,
           ba_msg_W1, ba_msg_b1, ba_msg_W2, ba_msg_b2,
           ba_udt_W1, ba_udt_b1, ba_udt_W2, ba_udt_b2):
    src_ab = edge_index_ab[0].astype(jnp.int32)
    dst_ab = edge_index_ab[1].astype(jnp.int32)
    src_ba = edge_index_ba[0].astype(jnp.int32)
    dst_ba = edge_index_ba[1].astype(jnp.int32)

    ab_W1e, ab_W1x = ab_msg_W1[:, :D_EDGE], ab_msg_W1[:, D_EDGE:]
    ba_W1e, ba_W1x = ba_msg_W1[:, :D_EDGE], ba_msg_W1[:, D_EDGE:]
    ab_uW1x, ab_uW1r = ab_udt_W1[:, :D_IN], ab_udt_W1[:, D_IN:]
    ba_uW1x, ba_uW1r = ba_udt_W1[:, :D_IN], ba_udt_W1[:, D_IN:]

    g_ab, g_ba = _node_proj(x_A, x_B, ab_W1x, ab_msg_b1, ba_W1x, ba_msg_b1)
    gat_ab, gat_ba = _sc_gather(g_ab, src_ab, g_ba, src_ba)

    mt_ab = _msg_mlp(edge_feat_ab, gat_ab, ab_W1e, ab_msg_W2, ab_msg_b2)
    mt_ba = _msg_mlp(edge_feat_ba, gat_ba, ba_W1e, ba_msg_W2, ba_msg_b2)

    r_ab, r_ba = _sc_segment_max(mt_ab, dst_ab, mt_ba, dst_ba)

    out_B = _udt_mlp(x_B, r_ab, ab_uW1x, ab_uW1r, ab_udt_b1,
                     ab_udt_W2, ab_udt_b2)
    out_A = _udt_mlp(x_A, r_ba, ba_uW1x, ba_uW1r, ba_udt_b1,
                     ba_udt_W2, ba_udt_b2)
    return (out_A, out_B)
